# Initial kernel scaffold; baseline (speedup 1.0000x reference)
import jax
import jax.numpy as jnp
from jax.experimental import pallas as pl


def kernel(x, edge_index, edge_weight, W1_0, b1_0, W2_0, b2_0, W1_1, b1_1, W2_1, b2_1, W1_2, b1_2, W2_2, b2_2):
    raise NotImplementedError("write your pallas kernel here")


# speedup vs baseline: 4.9558x; 4.9558x over previous
import functools

import jax
import jax.numpy as jnp
from jax import lax
from jax.experimental import pallas as pl
from jax.experimental.pallas import tpu as pltpu
from jax.experimental.pallas import tpu_sc as plsc

_N = 10000
_E = 320000
_D = 128
_CH = 128
_NCHUNK = _E // _CH
_NC = 2
_NS = 16
_NW = _NC * _NS
_STRIPE = 624
_LAST_STRIPE = _N - 15 * _STRIPE
_ITERS = (_NCHUNK + _NW - 1) // _NW


def _make_side_kernel(real_d):
  mesh = plsc.VectorSubcoreMesh(core_axis_name="c", subcore_axis_name="s",
                                num_cores=_NC, num_subcores=_NS)

  def body(ego, srcm, dstm, wm, zeros_hbm, out, acc, src_v, dst_v, w_v, rows,
           sem):
    cid = lax.axis_index("c")
    sid = lax.axis_index("s")
    wid = sid * _NC + cid
    r0 = pl.multiple_of(sid * _STRIPE, 8)

    @pl.when(sid < _NS - 1)
    def _():
      pltpu.sync_copy(zeros_hbm.at[pl.ds(r0, _STRIPE)],
                      acc.at[pl.ds(r0, _STRIPE)])

    @pl.when(sid == _NS - 1)
    def _():
      pltpu.sync_copy(zeros_hbm.at[pl.ds(15 * _STRIPE, _LAST_STRIPE)],
                      acc.at[pl.ds(15 * _STRIPE, _LAST_STRIPE)])

    plsc.subcore_barrier()

    def chunk_body(i, carry):
      c = wid + _NW * i

      @pl.when(c < _NCHUNK)
      def _():
        off = pl.multiple_of(c * _CH, _CH)
        pltpu.sync_copy(srcm.at[pl.ds(off, _CH)], src_v)
        pltpu.sync_copy(dstm.at[pl.ds(off, _CH)], dst_v)
        pltpu.sync_copy(wm.at[pl.ds(off, _CH)], w_v)
        pltpu.async_copy(ego.at[src_v], rows, sem).wait()

        def scale_body(t, c2):
          w16 = w_v[pl.ds(t * 16, 16)]
          for l in range(16):
            ws = w16[l]
            e = t * 16 + l
            for j in range(real_d // 16):
              sl = pl.ds(j * 16, 16)
              rows[e, sl] = rows[e, sl] * ws
          return c2

        lax.fori_loop(0, _CH // 16, scale_body, 0)

        pltpu.sync_copy(rows, acc.at[dst_v], add=True)

      return carry

    lax.fori_loop(0, _ITERS, chunk_body, 0)
    plsc.subcore_barrier()

    @pl.when(sid < _NS - 1)
    def _():
      pltpu.sync_copy(acc.at[pl.ds(r0, _STRIPE)],
                      out.at[cid, pl.ds(r0, _STRIPE)])

    @pl.when(sid == _NS - 1)
    def _():
      pltpu.sync_copy(acc.at[pl.ds(15 * _STRIPE, _LAST_STRIPE)],
                      out.at[cid, pl.ds(15 * _STRIPE, _LAST_STRIPE)])

  return pl.kernel(
      body,
      out_type=jax.ShapeDtypeStruct((_NC, _N, _D), jnp.float32),
      mesh=mesh,
      scratch_types=[
          pltpu.VMEM_SHARED((_N, _D), jnp.float32),
          pltpu.VMEM((_CH,), jnp.int32),
          pltpu.VMEM((_CH,), jnp.int32),
          pltpu.VMEM((_CH,), jnp.float32),
          pltpu.VMEM((_CH, _D), jnp.float32),
          pltpu.SemaphoreType.DMA,
      ],
  )


def _make_dense_kernel(Do, blk):
  def body(ego_ref, p0_ref, p1_ref, w1_ref, b1_ref, w2_ref, b2_ref,
           eg_ref, nm_ref):
    ego = ego_ref[...]
    side = p0_ref[...] + p1_ref[...]
    h1 = jnp.dot(ego + side, w1_ref[...],
                 preferred_element_type=jnp.float32) + b1_ref[...]
    h1 = jnp.where(h1 >= 0, h1, 0.01 * h1)
    h2 = jnp.dot(ego * side, w2_ref[...],
                 preferred_element_type=jnp.float32) + b2_ref[...]
    h2 = jnp.where(h2 >= 0, h2, 0.01 * h2)
    eg = h1 + h2
    eg_ref[...] = jnp.concatenate(
        [eg, jnp.zeros((eg.shape[0], _D - Do), jnp.float32)], axis=1)
    nrm = jnp.sqrt(jnp.sum(eg * eg, axis=1, keepdims=True))
    nm_ref[...] = eg / jnp.maximum(nrm, 1e-12)

  return pl.pallas_call(
      body,
      grid=(_N // blk,),
      in_specs=[
          pl.BlockSpec((blk, _D), lambda i: (i, 0)),
          pl.BlockSpec((blk, _D), lambda i: (i, 0)),
          pl.BlockSpec((blk, _D), lambda i: (i, 0)),
          pl.BlockSpec((_D, Do), lambda i: (0, 0)),
          pl.BlockSpec((1, Do), lambda i: (0, 0)),
          pl.BlockSpec((_D, Do), lambda i: (0, 0)),
          pl.BlockSpec((1, Do), lambda i: (0, 0)),
      ],
      out_specs=[
          pl.BlockSpec((blk, _D), lambda i: (i, 0)),
          pl.BlockSpec((blk, Do), lambda i: (i, 0)),
      ],
      out_shape=[
          jax.ShapeDtypeStruct((_N, _D), jnp.float32),
          jax.ShapeDtypeStruct((_N, Do), jnp.float32),
      ],
  )


_DIMS = [(128, 64), (64, 32), (32, 16)]
_SIDE = {D: _make_side_kernel(D) for D, _ in _DIMS}
_DENSE = {Do: _make_dense_kernel(Do, 2000) for _, Do in _DIMS}


def kernel(x, edge_index, edge_weight, W1_0, b1_0, W2_0, b2_0, W1_1, b1_1,
           W2_1, b2_1, W1_2, b1_2, W2_2, b2_2):
  src = edge_index[0]
  dst = edge_index[1]
  wm = edge_weight
  params = [(W1_0, b1_0, W2_0, b2_0), (W1_1, b1_1, W2_1, b2_1),
            (W1_2, b1_2, W2_2, b2_2)]
  zeros = jnp.zeros((_N, _D), jnp.float32)
  ego = x
  outs = [x]
  for (W1, b1, W2, b2), (D, Do) in zip(params, _DIMS):
    W1p = jnp.pad(W1, ((0, _D - D), (0, 0)))
    W2p = jnp.pad(W2, ((0, _D - D), (0, 0)))
    parts = _SIDE[D](ego, src, dst, wm, zeros)
    eg, nm = _DENSE[Do](ego, parts[0], parts[1], W1p, b1.reshape(1, Do),
                        W2p, b2.reshape(1, Do))
    ego = eg
    outs.append(nm)
  return jnp.concatenate(outs, axis=1)
